# baseline (speedup 1.0000x reference)
import functools

import jax
import jax.numpy as jnp
from jax import lax
from jax.experimental import pallas as pl
from jax.experimental.pallas import tpu as pltpu
from jax.experimental.pallas import tpu_sc as plsc

_L = 128
_D = 64
_H = 50
_NT = 128


@functools.lru_cache(maxsize=None)
def _build(n_rows: int):
    info = plsc.get_sparse_core_info()
    nc, ns = info.num_cores, info.num_subcores
    nw = nc * ns

    n_slabs = _H * _NT
    per_w = n_slabs // nw

    mesh = plsc.VectorSubcoreMesh(core_axis_name="c", subcore_axis_name="s")

    @functools.partial(
        pl.kernel,
        mesh=mesh,
        compiler_params=pltpu.CompilerParams(
            use_tc_tiling_on_sc=True, needs_layout_passes=False
        ),
        out_type=jax.ShapeDtypeStruct((_H, _D, _NT * _L), jnp.float32),
        scratch_types=[
            pltpu.VMEM((per_w, _L), jnp.int32),
            pltpu.VMEM((4, 8, _L), jnp.int32),
            pltpu.VMEM((4, _L, _L), jnp.float32),
            pltpu.VMEM((4, _D, _L), jnp.float32),
            pltpu.SemaphoreType.DMA,
            pltpu.SemaphoreType.DMA,
            pltpu.SemaphoreType.DMA,
            pltpu.SemaphoreType.DMA,
            pltpu.SemaphoreType.DMA,
            pltpu.SemaphoreType.DMA,
            pltpu.SemaphoreType.DMA,
            pltpu.SemaphoreType.DMA,
        ],
    )
    def gather_kernel(
        xr_hbm, tab_hbm, out_hbm, idx_v, id2_v, rec_v, slab_v,
        g0, g1, g2, g3, s0, s1, s2, s3,
    ):
        gsems = (g0, g1, g2, g3)
        ssems = (s0, s1, s2, s3)
        wid = lax.axis_index("s") * nc + lax.axis_index("c")
        s_base = wid * per_w
        pltpu.sync_copy(xr_hbm.at[pl.ds(s_base, per_w)], idx_v)

        iotas = [lax.iota(jnp.int32, 16) + 16 * seg for seg in range(8)]

        def prep_ids(sl, slot):
            for seg in range(8):
                v = idx_v[sl, pl.ds(seg * 16, 16)]
                id2_v[slot, 0, pl.ds(seg * 16, 16)] = v >> 1
                id2_v[slot, 1, pl.ds(seg * 16, 16)] = (v & 1) << 6

        def gather_desc(slot):
            return pltpu.make_async_copy(
                tab_hbm.at[id2_v.at[slot, 0]],
                rec_v.at[slot],
                gsems[slot],
            )

        def store_desc(s, slot):
            h = s // _NT
            t = s % _NT
            return pltpu.make_async_copy(
                slab_v.at[slot],
                out_hbm.at[h].at[:, pl.ds(t * _L, _L)],
                ssems[slot],
            )

        def transpose(slot):
            offs = [id2_v[slot, 1, pl.ds(seg * 16, 16)] for seg in range(8)]

            @plsc.parallel_loop(0, _D, step=1, unroll=16)
            def dbody(d):
                for seg in range(8):
                    vals = plsc.load_gather(
                        rec_v.at[slot], [iotas[seg], offs[seg] + d]
                    )
                    slab_v[slot, d, pl.ds(seg * 16, 16)] = vals

        for k in range(3):
            prep_ids(k, k)
            gather_desc(k).start()

        def body(g, carry):
            for k in range(4):
                sl = g * 4 + k
                nk = (k + 3) % 4
                gather_desc(k).wait()

                @pl.when(sl + 3 < per_w)
                def _():
                    prep_ids(sl + 3, nk)

                    @pl.when(sl >= 1)
                    def _():
                        store_desc(s_base + sl - 1, nk).wait()

                    gather_desc(nk).start()

                transpose(k)
                store_desc(s_base + sl, k).start()
            return carry

        lax.fori_loop(0, per_w // 4, body, 0)
        for k in range(4):
            store_desc(s_base + per_w - 4 + k, k).wait()

    return gather_kernel


def kernel(x, table):
    b, h = x.shape
    v, d = table.shape

    xr = jnp.transpose(x.astype(jnp.int32)).reshape(h * (b // _L), _L)
    tab2 = table.reshape(v // 2, 2 * d)

    gather_kernel = _build(xr.shape[0])
    out_t = gather_kernel(xr, tab2)
    return jnp.transpose(out_t, (2, 0, 1))


# speedup vs baseline: 1.0463x; 1.0463x over previous
import functools

import jax
import jax.numpy as jnp
from jax import lax
from jax.experimental import pallas as pl
from jax.experimental.pallas import tpu as pltpu
from jax.experimental.pallas import tpu_sc as plsc

_CHUNK = 128
_GRP = 4


@functools.lru_cache(maxsize=None)
def _build(n_chunks: int, d: int):
    info = plsc.get_sparse_core_info()
    nc, ns = info.num_cores, info.num_subcores
    nw = nc * ns

    mesh = plsc.VectorSubcoreMesh(core_axis_name="c", subcore_axis_name="s")
    per_w = n_chunks * _CHUNK
    n_groups = n_chunks // _GRP
    grows = _GRP * _CHUNK

    @functools.partial(
        pl.kernel,
        mesh=mesh,
        compiler_params=pltpu.CompilerParams(use_tc_tiling_on_sc=False),
        out_type=jax.ShapeDtypeStruct((nw * per_w, d), jnp.float32),
        scratch_types=[
            pltpu.VMEM((n_chunks, _CHUNK), jnp.int32),
            pltpu.VMEM((2, grows, d), jnp.float32),
            pltpu.SemaphoreType.DMA,
            pltpu.SemaphoreType.DMA,
            pltpu.SemaphoreType.DMA,
            pltpu.SemaphoreType.DMA,
        ],
    )
    def gather_kernel(idx_hbm, table_hbm, out_hbm, idx_v, rows_v, g0, g1, s0, s1):
        gsems = (g0, g1)
        ssems = (s0, s1)
        wid = lax.axis_index("s") * nc + lax.axis_index("c")
        base = wid * per_w
        pltpu.sync_copy(idx_hbm.at[wid], idx_v)

        def gather_desc(g, slot):
            return [
                pltpu.make_async_copy(
                    table_hbm.at[idx_v.at[g * _GRP + c]],
                    rows_v.at[slot].at[pl.ds(c * _CHUNK, _CHUNK)],
                    gsems[slot],
                )
                for c in range(_GRP)
            ]

        def store_desc(g, slot):
            return pltpu.make_async_copy(
                rows_v.at[slot],
                out_hbm.at[pl.ds(base + g * grows, grows)],
                ssems[slot],
            )

        for c in gather_desc(0, 0):
            c.start()

        def body(g2, carry):
            for s in (0, 1):
                g = g2 * 2 + s
                o = 1 - s
                for c in gather_desc(g, s):
                    c.wait()

                @pl.when(g + 1 < n_groups)
                def _():
                    @pl.when(g >= 1)
                    def _():
                        store_desc(g - 1, o).wait()

                    for c in gather_desc(g + 1, o):
                        c.start()

                store_desc(g, s).start()
            return carry

        lax.fori_loop(0, n_groups // 2, body, 0)
        store_desc(n_groups - 2, (n_groups - 2) % 2).wait()
        store_desc(n_groups - 1, (n_groups - 1) % 2).wait()

    return gather_kernel, nw


def kernel(x, table):
    b, h = x.shape
    v, d = table.shape
    n = b * h

    info = plsc.get_sparse_core_info()
    nw = info.num_cores * info.num_subcores
    tile = nw * _CHUNK * _GRP * 2
    n_pad = ((n + tile - 1) // tile) * tile

    x_flat = x.reshape(n).astype(jnp.int32)
    if n_pad != n:
        x_flat = jnp.concatenate([x_flat, jnp.zeros(n_pad - n, jnp.int32)])
    n_chunks = n_pad // (nw * _CHUNK)

    gather_kernel, nw = _build(n_chunks, d)
    out = gather_kernel(x_flat.reshape(nw, n_chunks, _CHUNK), table)
    return out[:n].reshape(b, h, d)
